# Initial kernel scaffold; baseline (speedup 1.0000x reference)
import jax
import jax.numpy as jnp
from jax.experimental import pallas as pl


def kernel(x, edge_index, edge_weight, conv_w0, conv_b0, conv_w1, conv_b1, conv_w2, conv_b2, w1, w2, fc1_w, fc1_b, fc2_w, fc2_b):
    raise NotImplementedError("write your pallas kernel here")


# speedup vs baseline: 5.0363x; 5.0363x over previous
import functools

import jax
import jax.numpy as jnp
from jax import lax
from jax.experimental import pallas as pl
from jax.experimental.pallas import tpu as pltpu
from jax.experimental.pallas import tpu_sc as plsc

_F = 128
_LANES = 16
_NVR = _F // _LANES
_CHUNK = 128
_NTILES = 16
_ROWBLK = 2000


def _leaky(v):
    return jnp.maximum(v, 0.2 * v)


def _dot_t(a, w):
    return lax.dot_general(a, w, (((1,), (1,)), ((), ())),
                           preferred_element_type=jnp.float32)


def _tc_linear(w, b, s, hl=None, pre_act=False):
    m = s.shape[0]

    def body(*refs):
        if hl is None:
            s_ref, w_ref, b_ref, o_ref = refs
            a = s_ref[...]
            if pre_act:
                a = _leaky(a)
        else:
            s_ref, hl_ref, w_ref, b_ref, o_ref = refs
            a = _leaky(s_ref[...] + hl_ref[...])
        o_ref[...] = _dot_t(a, w_ref[...]) + b_ref[...]

    in_specs = [pl.BlockSpec((_ROWBLK, _F), lambda i: (i, 0))]
    args = [s]
    if hl is not None:
        in_specs.append(pl.BlockSpec((_ROWBLK, _F), lambda i: (i, 0)))
        args.append(hl)
    in_specs += [pl.BlockSpec((_F, _F), lambda i: (0, 0)),
                 pl.BlockSpec((_F,), lambda i: (0,))]
    args += [w, b]
    return pl.pallas_call(
        body,
        grid=(m // _ROWBLK,),
        in_specs=in_specs,
        out_specs=pl.BlockSpec((_ROWBLK, _F), lambda i: (i, 0)),
        out_shape=jax.ShapeDtypeStruct((m, _F), jnp.float32),
    )(*args)


def _tc_head(s2, hl2, w1, w2, fc1_w, fc1_b, fc2_w, fc2_b, n):
    nb = n // _ROWBLK
    f2 = 2 * _F

    def body(sa, ha, sb, hb, w1r, w2r, f1w, f1b, f2w, f2b, y_ref, xo_ref):
        a1 = _leaky(sa[...] + ha[...])
        a2 = _leaky(sb[...] + hb[...])
        xc = jnp.concatenate([_dot_t(a1, w1r[...]), _dot_t(a2, w2r[...])],
                             axis=1)
        y = _dot_t(xc, f1w[...]) + f1b[...]
        y_ref[...] = y
        z = jnp.sum(y * f2w[...], axis=1) + f2b[...]
        xo_ref[...] = _leaky(z)[:, None]

    half_specs = [
        pl.BlockSpec((_ROWBLK, _F), lambda i: (i, 0)),
        pl.BlockSpec((_ROWBLK, _F), lambda i: (i, 0)),
        pl.BlockSpec((_ROWBLK, _F), lambda i: (i + nb, 0)),
        pl.BlockSpec((_ROWBLK, _F), lambda i: (i + nb, 0)),
    ]
    w_specs = [
        pl.BlockSpec((_F, _F), lambda i: (0, 0)),
        pl.BlockSpec((_F, _F), lambda i: (0, 0)),
        pl.BlockSpec((f2, f2), lambda i: (0, 0)),
        pl.BlockSpec((f2,), lambda i: (0,)),
        pl.BlockSpec((1, f2), lambda i: (0, 0)),
        pl.BlockSpec((1,), lambda i: (0,)),
    ]
    y, xo = pl.pallas_call(
        body,
        grid=(nb,),
        in_specs=half_specs + w_specs,
        out_specs=[pl.BlockSpec((_ROWBLK, f2), lambda i: (i, 0)),
                   pl.BlockSpec((_ROWBLK, 1), lambda i: (i, 0))],
        out_shape=[jax.ShapeDtypeStruct((n, f2), jnp.float32),
                   jax.ShapeDtypeStruct((n, 1), jnp.float32)],
    )(s2, hl2, s2, hl2, w1, w2, fc1_w, fc1_b, fc2_w, fc2_b)
    return y, xo.reshape(-1)


def _sc_propagate(hl, src, dst, ew, n):
    epad = src.shape[0]
    e_per_tile = epad // _NTILES
    nchunks = e_per_tile // _CHUNK
    stripe = (n // _NTILES) // 8 * 8
    rem = n - stripe * _NTILES
    mesh = plsc.VectorSubcoreMesh(core_axis_name="c", subcore_axis_name="s")

    @functools.partial(
        pl.kernel,
        out_type=jax.ShapeDtypeStruct((2 * n, _F), jnp.float32),
        mesh=mesh,
        scratch_types=[
            pltpu.VMEM((_CHUNK,), jnp.int32),
            pltpu.VMEM((_CHUNK,), jnp.int32),
            pltpu.VMEM((_CHUNK,), jnp.float32),
            pltpu.VMEM((_CHUNK,), jnp.int32),
            pltpu.VMEM((_CHUNK,), jnp.int32),
            pltpu.VMEM((_CHUNK, _F), jnp.float32),
            pltpu.VMEM_SHARED((n, _F), jnp.float32),
            pltpu.SemaphoreType.DMA,
        ],
    )
    def k(hl_hbm, src_hbm, dst_hbm, ew_hbm, out_hbm,
          srcb, dstb, ewb, gidx, sidx, rows, acc, sem):
        c = lax.axis_index("c")
        s = lax.axis_index("s")

        def zbody(r, carry):
            for v in range(_NVR):
                rows[r, pl.ds(v * _LANES, _LANES)] = jnp.zeros(
                    (_LANES,), jnp.float32)
            return carry
        lax.fori_loop(0, _CHUNK, zbody, 0)
        base_row = s * stripe
        nfull, tail = divmod(stripe, _CHUNK)
        for j in range(nfull):
            pltpu.sync_copy(rows.at[pl.ds(0, _CHUNK)],
                            acc.at[pl.ds(base_row + j * _CHUNK, _CHUNK)])
        if tail:
            pltpu.sync_copy(rows.at[pl.ds(0, tail)],
                            acc.at[pl.ds(base_row + nfull * _CHUNK, tail)])
        if rem:
            @pl.when(s == 0)
            def _():
                pltpu.sync_copy(rows.at[pl.ds(0, rem)],
                                acc.at[pl.ds(stripe * _NTILES, rem)])
        plsc.subcore_barrier()

        ebase = s * e_per_tile

        def chunk_body(j, carry):
            off = ebase + j * _CHUNK
            pltpu.sync_copy(src_hbm.at[pl.ds(off, _CHUNK)], srcb)
            pltpu.sync_copy(dst_hbm.at[pl.ds(off, _CHUNK)], dstb)
            pltpu.sync_copy(ew_hbm.at[pl.ds(off, _CHUNK)], ewb)
            for v in range(_CHUNK // _LANES):
                sl = pl.ds(v * _LANES, _LANES)
                sv = srcb[sl]
                dv = dstb[sl]
                gidx[sl] = dv + (sv - dv + n) * c
                sidx[sl] = sv + (dv - sv) * c
            pltpu.async_copy(hl_hbm.at[gidx], rows, sem).wait()

            def scale_body(g, cc):
                wv = ewb[pl.ds(g * _LANES, _LANES)]
                for l in range(_LANES):
                    w = wv[l]
                    e = g * _LANES + l
                    for v in range(_NVR):
                        sl = pl.ds(v * _LANES, _LANES)
                        rows[e, sl] = rows[e, sl] * w
                return cc
            lax.fori_loop(0, _CHUNK // _LANES, scale_body, 0)
            pltpu.sync_copy(rows, acc.at[sidx], add=True)
            return carry
        lax.fori_loop(0, nchunks, chunk_body, 0)

        plsc.subcore_barrier()
        pltpu.sync_copy(acc.at[pl.ds(base_row, stripe)],
                        out_hbm.at[pl.ds(c * n + base_row, stripe)])
        if rem:
            @pl.when(s == 0)
            def _():
                pltpu.sync_copy(
                    acc.at[pl.ds(stripe * _NTILES, rem)],
                    out_hbm.at[pl.ds(c * n + stripe * _NTILES, rem)])

    return k(hl, src, dst, ew)


def kernel(x, edge_index, edge_weight, conv_w0, conv_b0, conv_w1, conv_b1,
           conv_w2, conv_b2, w1, w2, fc1_w, fc1_b, fc2_w, fc2_b):
    n = x.shape[0]
    f = x.shape[1] // 2
    h0 = jnp.concatenate([x[:, f:], x[:, :f]], axis=0)

    src = edge_index[0]
    dst = edge_index[1]
    e = src.shape[0]
    per_tile = -(-e // _NTILES)
    per_tile = -(-per_tile // _CHUNK) * _CHUNK
    pad = per_tile * _NTILES - e
    srcp = jnp.concatenate([src, jnp.zeros((pad,), jnp.int32)])
    dstp = jnp.concatenate([dst, jnp.zeros((pad,), jnp.int32)])
    ewp = jnp.concatenate([edge_weight, jnp.zeros((pad,), jnp.float32)])

    hl0 = _tc_linear(conv_w0, conv_b0, h0)
    s0 = _sc_propagate(hl0, srcp, dstp, ewp, n)
    hl1 = _tc_linear(conv_w1, conv_b1, s0, pre_act=True)
    s1 = _sc_propagate(hl1, srcp, dstp, ewp, n)
    hl2 = _tc_linear(conv_w2, conv_b2, s1, hl=hl1)
    s2 = _sc_propagate(hl2, srcp, dstp, ewp, n)
    y, xo = _tc_head(s2, hl2, w1, w2, fc1_w, fc1_b, fc2_w, fc2_b, n)
    return (xo, y)
